# Initial kernel scaffold; baseline (speedup 1.0000x reference)
import jax
import jax.numpy as jnp
from jax.experimental import pallas as pl


def kernel(x, edges, layers_lengths, W1, b1, W2, b2, W3, b3, Wc, bc):
    raise NotImplementedError("write your pallas kernel here")


# speedup vs baseline: 4.1870x; 4.1870x over previous
import functools

import jax
import jax.numpy as jnp
from jax import lax
from jax.experimental import pallas as pl
from jax.experimental.pallas import tpu as pltpu
from jax.experimental.pallas import tpu_sc as plsc

N = 10000
NP = 10240
E = 160000
EP = 163840
ER = EP // 128
NC, NS = 2, 16
RPT = NP // NS
BN = 256

_mesh = plsc.VectorSubcoreMesh(core_axis_name="c", subcore_axis_name="s")
_sc_params = pltpu.CompilerParams(use_tc_tiling_on_sc=False)


def _deg(dst3):
    @functools.partial(
        pl.kernel,
        out_type=jax.ShapeDtypeStruct((NC, NP, 16), jnp.float32),
        mesh=_mesh,
        compiler_params=_sc_params,
        scratch_types=[
            pltpu.VMEM((4, 128), jnp.int32),
            pltpu.VMEM((128, 16), jnp.float32),
            pltpu.VMEM((128, 16), jnp.float32),
            pltpu.VMEM_SHARED((NP, 16), jnp.float32),
        ],
    )
    def k(dst_hbm, deg_hbm, idx_v, ones_v, zero_v, deg_sh):
        core = lax.axis_index("c")
        sid = lax.axis_index("s")

        @pl.loop(0, 128)
        def _(i):
            ones_v[i] = jnp.full((16,), 1.0, jnp.float32)
            zero_v[i] = jnp.zeros((16,), jnp.float32)

        @pl.loop(0, RPT // 128)
        def _(i):
            pltpu.sync_copy(zero_v, deg_sh.at[pl.ds(sid * RPT + i * 128, 128)])

        plsc.subcore_barrier()

        rpt = ER // NC // NS

        @pl.loop(0, rpt // 4)
        def _(r):
            base = core * (ER // NC) + sid * rpt + r * 4
            pltpu.sync_copy(dst_hbm.at[pl.ds(base, 4)], idx_v)
            for j in range(4):
                pltpu.sync_copy(ones_v, deg_sh.at[idx_v.at[j]], add=True)

        plsc.subcore_barrier()

        @pl.loop(0, RPT // 128)
        def _(i):
            s = pl.ds(sid * RPT + i * 128, 128)
            pltpu.sync_copy(deg_sh.at[s], deg_hbm.at[core].at[s])

    return k(dst3)


def _propagate(hs, src3, dst3, C, Dc):
    @functools.partial(
        pl.kernel,
        out_type=jax.ShapeDtypeStruct((C, NP, Dc), jnp.float32),
        mesh=_mesh,
        compiler_params=_sc_params,
        scratch_types=[
            pltpu.VMEM((4, 128), jnp.int32),
            pltpu.VMEM((4, 128), jnp.int32),
            pltpu.VMEM((512, Dc), jnp.float32),
            pltpu.VMEM((128, Dc), jnp.float32),
            pltpu.VMEM_SHARED((NP, Dc), jnp.float32),
        ],
    )
    def k(hs_hbm, src_hbm, dst_hbm, acc_hbm, src_v, dst_v, rows_v, zero_v, acc_sh):
        core = lax.axis_index("c")
        sid = lax.axis_index("s")

        @pl.loop(0, 128)
        def _(i):
            for j in range(Dc // 16):
                zero_v[i, pl.ds(j * 16, 16)] = jnp.zeros((16,), jnp.float32)

        for cc in range(C // NC):
            c = cc * NC + core

            @pl.loop(0, RPT // 128)
            def _(i):
                pltpu.sync_copy(zero_v, acc_sh.at[pl.ds(sid * RPT + i * 128, 128)])

            plsc.subcore_barrier()

            rpt = ER // NS

            @pl.loop(0, rpt // 4)
            def _(r):
                base = sid * rpt + r * 4
                pltpu.sync_copy(src_hbm.at[pl.ds(base, 4)], src_v)
                pltpu.sync_copy(dst_hbm.at[pl.ds(base, 4)], dst_v)
                for j in range(4):
                    pltpu.sync_copy(hs_hbm.at[c].at[src_v.at[j]],
                                    rows_v.at[pl.ds(j * 128, 128)])
                for j in range(4):
                    pltpu.sync_copy(rows_v.at[pl.ds(j * 128, 128)],
                                    acc_sh.at[dst_v.at[j]], add=True)

            plsc.subcore_barrier()

            @pl.loop(0, RPT // 128)
            def _(i):
                s = pl.ds(sid * RPT + i * 128, 128)
                pltpu.sync_copy(acc_sh.at[s], acc_hbm.at[c].at[s])

    return k(hs, src3, dst3)


def _matmul(xp, Wp):
    K, Do = Wp.shape

    def body(x_ref, w_ref, o_ref):
        o_ref[...] = jnp.dot(x_ref[...], w_ref[...],
                             preferred_element_type=jnp.float32)

    return pl.pallas_call(
        body,
        grid=(NP // BN,),
        in_specs=[pl.BlockSpec((BN, K), lambda i: (i, 0)),
                  pl.BlockSpec((K, Do), lambda i: (0, 0))],
        out_specs=pl.BlockSpec((BN, Do), lambda i: (i, 0)),
        out_shape=jax.ShapeDtypeStruct((NP, Do), jnp.float32),
    )(xp, Wp)


def _scale_chunk(h1, deg16):
    def body(h_ref, deg_ref, hs_ref, dinv_ref):
        dg = deg_ref[...]
        deg = dg[0, :, 0] + dg[1, :, 0] + 1.0
        dinv = lax.rsqrt(deg)
        dinv_ref[...] = dinv
        hs = h_ref[...] * dinv[:, None]
        for ci in range(8):
            hs_ref[ci] = hs[:, ci * 64:(ci + 1) * 64]

    return pl.pallas_call(
        body,
        grid=(NP // BN,),
        in_specs=[pl.BlockSpec((BN, 512), lambda i: (i, 0)),
                  pl.BlockSpec((2, BN, 16), lambda i: (0, i, 0))],
        out_specs=[pl.BlockSpec((8, BN, 64), lambda i: (0, i, 0)),
                   pl.BlockSpec((BN,), lambda i: (i,))],
        out_shape=[jax.ShapeDtypeStruct((8, NP, 64), jnp.float32),
                   jax.ShapeDtypeStruct((NP,), jnp.float32)],
    )(h1, deg16)


def _trans(acc, hs, dinv, b, W, C, Dc, C2, Dc2):
    dk = C * Dc
    dn = W.shape[1]

    def body(acc_ref, hs_ref, dinv_ref, b_ref, w_ref, o_ref):
        dinv = dinv_ref[...]
        s = None
        for ci in range(C):
            z = jnp.clip(dinv[:, None] * (acc_ref[ci] + hs_ref[ci])
                         + b_ref[ci * Dc:(ci + 1) * Dc], 0.0, 6.0)
            p = jnp.dot(z, w_ref[ci * Dc:(ci + 1) * Dc, :],
                        preferred_element_type=jnp.float32)
            s = p if s is None else s + p
        hsn = s * dinv[:, None]
        for ci in range(C2):
            o_ref[ci] = hsn[:, ci * Dc2:(ci + 1) * Dc2]

    return pl.pallas_call(
        body,
        grid=(NP // BN,),
        in_specs=[pl.BlockSpec((C, BN, Dc), lambda i: (0, i, 0)),
                  pl.BlockSpec((C, BN, Dc), lambda i: (0, i, 0)),
                  pl.BlockSpec((BN,), lambda i: (i,)),
                  pl.BlockSpec((dk,), lambda i: (0,)),
                  pl.BlockSpec((dk, dn), lambda i: (0, 0))],
        out_specs=pl.BlockSpec((C2, BN, Dc2), lambda i: (0, i, 0)),
        out_shape=jax.ShapeDtypeStruct((C2, NP, Dc2), jnp.float32),
    )(acc, hs, dinv, b, W)


def _final(acc3, hs3, dinv, b3p, Wcp, bcp):
    def body(acc_ref, hs_ref, dinv_ref, b_ref, wc_ref, bc_ref, out_ref, h_ref):
        dinv = dinv_ref[...]
        s = None
        for ci in range(2):
            z = jnp.clip(dinv[:, None] * (acc_ref[ci] + hs_ref[ci])
                         + b_ref[ci * 32:(ci + 1) * 32], 0.0, 6.0)
            h_ref[:, ci * 32:(ci + 1) * 32] = z
            p = jnp.dot(z, wc_ref[ci * 32:(ci + 1) * 32, :],
                        preferred_element_type=jnp.float32)
            s = p if s is None else s + p
        out_ref[...] = jax.nn.sigmoid(s + bc_ref[...])

    return pl.pallas_call(
        body,
        grid=(NP // BN,),
        in_specs=[pl.BlockSpec((2, BN, 32), lambda i: (0, i, 0)),
                  pl.BlockSpec((2, BN, 32), lambda i: (0, i, 0)),
                  pl.BlockSpec((BN,), lambda i: (i,)),
                  pl.BlockSpec((64,), lambda i: (0,)),
                  pl.BlockSpec((64, 128), lambda i: (0, 0)),
                  pl.BlockSpec((128,), lambda i: (0,))],
        out_specs=[pl.BlockSpec((BN, 128), lambda i: (i, 0)),
                   pl.BlockSpec((BN, 64), lambda i: (i, 0))],
        out_shape=[jax.ShapeDtypeStruct((NP, 128), jnp.float32),
                   jax.ShapeDtypeStruct((NP, 64), jnp.float32)],
    )(acc3, hs3, dinv, b3p, Wcp, bcp)


def kernel(x, edges, layers_lengths, W1, b1, W2, b2, W3, b3, Wc, bc):
    del layers_lengths
    f32 = jnp.float32
    pad = jnp.full((EP - E,), N, jnp.int32)
    src3 = jnp.concatenate([edges[0], pad]).reshape(ER, 128)
    dst3 = jnp.concatenate([edges[1], pad]).reshape(ER, 128)

    xp = jnp.zeros((NP, 1024), f32).at[:N, :1000].set(x)
    W1p = jnp.zeros((1024, 512), f32).at[:1000].set(W1)
    W3p = jnp.zeros((256, 64), f32).at[:, :52].set(W3)
    b3p = jnp.zeros((64,), f32).at[:52].set(b3)
    Wcp = jnp.zeros((64, 128), f32).at[:52, :3].set(Wc)
    bcp = jnp.zeros((128,), f32).at[:3].set(bc)

    deg16 = _deg(dst3)
    h1 = _matmul(xp, W1p)
    hs1, dinv = _scale_chunk(h1, deg16)
    acc1 = _propagate(hs1, src3, dst3, 8, 64)
    hs2 = _trans(acc1, hs1, dinv, b1, W2, 8, 64, 4, 64)
    acc2 = _propagate(hs2, src3, dst3, 4, 64)
    hs3 = _trans(acc2, hs2, dinv, b2, W3p, 4, 64, 2, 32)
    acc3 = _propagate(hs3, src3, dst3, 2, 32)
    out_full, h_full = _final(acc3, hs3, dinv, b3p, Wcp, bcp)
    return (out_full[:N, :3], h_full[:N, :52])


# speedup vs baseline: 5.6523x; 1.3500x over previous
import functools

import jax
import jax.numpy as jnp
from jax import lax
from jax.experimental import pallas as pl
from jax.experimental.pallas import tpu as pltpu
from jax.experimental.pallas import tpu_sc as plsc

N = 10000
NP = 10240
E = 160000
EP = 163840
ER = EP // 128
NC, NS = 2, 16
RPT = NP // NS
BN = 256

_mesh = plsc.VectorSubcoreMesh(core_axis_name="c", subcore_axis_name="s")
_sc_params = pltpu.CompilerParams(use_tc_tiling_on_sc=False)


def _deg(dst3):
    @functools.partial(
        pl.kernel,
        out_type=jax.ShapeDtypeStruct((NC, NP, 16), jnp.float32),
        mesh=_mesh,
        compiler_params=_sc_params,
        scratch_types=[
            pltpu.VMEM((4, 128), jnp.int32),
            pltpu.VMEM((128, 16), jnp.float32),
            pltpu.VMEM((128, 16), jnp.float32),
            pltpu.VMEM_SHARED((NP, 16), jnp.float32),
        ],
    )
    def k(dst_hbm, deg_hbm, idx_v, ones_v, zero_v, deg_sh):
        core = lax.axis_index("c")
        sid = lax.axis_index("s")

        @pl.loop(0, 128)
        def _(i):
            ones_v[i] = jnp.full((16,), 1.0, jnp.float32)
            zero_v[i] = jnp.zeros((16,), jnp.float32)

        @pl.loop(0, RPT // 128)
        def _(i):
            pltpu.sync_copy(zero_v, deg_sh.at[pl.ds(sid * RPT + i * 128, 128)])

        plsc.subcore_barrier()

        rpt = ER // NC // NS

        @pl.loop(0, rpt // 4)
        def _(r):
            base = core * (ER // NC) + sid * rpt + r * 4
            pltpu.sync_copy(dst_hbm.at[pl.ds(base, 4)], idx_v)
            for j in range(4):
                pltpu.sync_copy(ones_v, deg_sh.at[idx_v.at[j]], add=True)

        plsc.subcore_barrier()

        @pl.loop(0, RPT // 128)
        def _(i):
            s = pl.ds(sid * RPT + i * 128, 128)
            pltpu.sync_copy(deg_sh.at[s], deg_hbm.at[core].at[s])

    return k(dst3)


NBUF = 4


def _propagate(hs, src3, dst3, C, Dc):
    rpt = ER // NS

    @functools.partial(
        pl.kernel,
        out_type=jax.ShapeDtypeStruct((C, NP, Dc), jnp.float32),
        mesh=_mesh,
        compiler_params=_sc_params,
        scratch_types=[
            pltpu.VMEM((rpt, 128), jnp.int32),
            pltpu.VMEM((rpt, 128), jnp.int32),
            pltpu.VMEM((NBUF, 128, Dc), jnp.float32),
            pltpu.VMEM((128, Dc), jnp.float32),
            pltpu.VMEM_SHARED((NP, Dc), jnp.float32),
        ] + [pltpu.SemaphoreType.DMA] * (2 * NBUF),
    )
    def k(hs_hbm, src_hbm, dst_hbm, acc_hbm, src_all, dst_all, ring, zero_v,
          acc_sh, *sems):
        gsem = sems[:NBUF]
        ssem = sems[NBUF:]
        core = lax.axis_index("c")
        sid = lax.axis_index("s")

        @pl.loop(0, 128)
        def _(i):
            for j in range(Dc // 16):
                zero_v[i, pl.ds(j * 16, 16)] = jnp.zeros((16,), jnp.float32)

        pltpu.sync_copy(src_hbm.at[pl.ds(sid * rpt, rpt)], src_all)
        pltpu.sync_copy(dst_hbm.at[pl.ds(sid * rpt, rpt)], dst_all)

        nloops = rpt // NBUF
        for cc in range(C // NC):
            c = cc * NC + core

            @pl.loop(0, RPT // 128)
            def _(i):
                pltpu.sync_copy(zero_v, acc_sh.at[pl.ds(sid * RPT + i * 128, 128)])

            plsc.subcore_barrier()

            for b in range(NBUF):
                pltpu.async_copy(hs_hbm.at[c].at[src_all.at[b]], ring.at[b],
                                 gsem[b])

            @pl.loop(0, nloops - 1)
            def _(r):
                for b in range(NBUF):
                    g = r * NBUF + b
                    pltpu.make_async_copy(hs_hbm.at[c].at[pl.ds(0, 128)],
                                          ring.at[b], gsem[b]).wait()
                    pltpu.async_copy(ring.at[b], acc_sh.at[dst_all.at[g]],
                                     ssem[b], add=True)
                    pltpu.make_async_copy(hs_hbm.at[c].at[pl.ds(0, 128)],
                                          ring.at[b], ssem[b]).wait()
                    pltpu.async_copy(hs_hbm.at[c].at[src_all.at[g + NBUF]],
                                     ring.at[b], gsem[b])

            for b in range(NBUF):
                g = (nloops - 1) * NBUF + b
                pltpu.make_async_copy(hs_hbm.at[c].at[pl.ds(0, 128)],
                                      ring.at[b], gsem[b]).wait()
                pltpu.async_copy(ring.at[b], acc_sh.at[dst_all.at[g]],
                                 ssem[b], add=True)
            for b in range(NBUF):
                pltpu.make_async_copy(hs_hbm.at[c].at[pl.ds(0, 128)],
                                      ring.at[b], ssem[b]).wait()

            plsc.subcore_barrier()
            pltpu.sync_copy(acc_sh.at[pl.ds(sid * RPT, RPT)],
                            acc_hbm.at[c].at[pl.ds(sid * RPT, RPT)])

    return k(hs, src3, dst3)


def _matmul(xp, Wp):
    K, Do = Wp.shape

    def body(x_ref, w_ref, o_ref):
        o_ref[...] = jnp.dot(x_ref[...], w_ref[...],
                             preferred_element_type=jnp.float32)

    return pl.pallas_call(
        body,
        grid=(NP // BN,),
        in_specs=[pl.BlockSpec((BN, K), lambda i: (i, 0)),
                  pl.BlockSpec((K, Do), lambda i: (0, 0))],
        out_specs=pl.BlockSpec((BN, Do), lambda i: (i, 0)),
        out_shape=jax.ShapeDtypeStruct((NP, Do), jnp.float32),
    )(xp, Wp)


def _scale_chunk(h1, deg16):
    def body(h_ref, deg_ref, hs_ref, dinv_ref):
        dg = deg_ref[...]
        deg = dg[0, :, 0] + dg[1, :, 0] + 1.0
        dinv = lax.rsqrt(deg)
        dinv_ref[...] = dinv
        hs = h_ref[...] * dinv[:, None]
        for ci in range(8):
            hs_ref[ci] = hs[:, ci * 64:(ci + 1) * 64]

    return pl.pallas_call(
        body,
        grid=(NP // BN,),
        in_specs=[pl.BlockSpec((BN, 512), lambda i: (i, 0)),
                  pl.BlockSpec((2, BN, 16), lambda i: (0, i, 0))],
        out_specs=[pl.BlockSpec((8, BN, 64), lambda i: (0, i, 0)),
                   pl.BlockSpec((BN,), lambda i: (i,))],
        out_shape=[jax.ShapeDtypeStruct((8, NP, 64), jnp.float32),
                   jax.ShapeDtypeStruct((NP,), jnp.float32)],
    )(h1, deg16)


def _trans(acc, hs, dinv, b, W, C, Dc, C2, Dc2):
    dk = C * Dc
    dn = W.shape[1]

    def body(acc_ref, hs_ref, dinv_ref, b_ref, w_ref, o_ref):
        dinv = dinv_ref[...]
        s = None
        for ci in range(C):
            z = jnp.clip(dinv[:, None] * (acc_ref[ci] + hs_ref[ci])
                         + b_ref[ci * Dc:(ci + 1) * Dc], 0.0, 6.0)
            p = jnp.dot(z, w_ref[ci * Dc:(ci + 1) * Dc, :],
                        preferred_element_type=jnp.float32)
            s = p if s is None else s + p
        hsn = s * dinv[:, None]
        for ci in range(C2):
            o_ref[ci] = hsn[:, ci * Dc2:(ci + 1) * Dc2]

    return pl.pallas_call(
        body,
        grid=(NP // BN,),
        in_specs=[pl.BlockSpec((C, BN, Dc), lambda i: (0, i, 0)),
                  pl.BlockSpec((C, BN, Dc), lambda i: (0, i, 0)),
                  pl.BlockSpec((BN,), lambda i: (i,)),
                  pl.BlockSpec((dk,), lambda i: (0,)),
                  pl.BlockSpec((dk, dn), lambda i: (0, 0))],
        out_specs=pl.BlockSpec((C2, BN, Dc2), lambda i: (0, i, 0)),
        out_shape=jax.ShapeDtypeStruct((C2, NP, Dc2), jnp.float32),
    )(acc, hs, dinv, b, W)


def _final(acc3, hs3, dinv, b3p, Wcp, bcp):
    def body(acc_ref, hs_ref, dinv_ref, b_ref, wc_ref, bc_ref, out_ref, h_ref):
        dinv = dinv_ref[...]
        s = None
        for ci in range(2):
            z = jnp.clip(dinv[:, None] * (acc_ref[ci] + hs_ref[ci])
                         + b_ref[ci * 32:(ci + 1) * 32], 0.0, 6.0)
            h_ref[:, ci * 32:(ci + 1) * 32] = z
            p = jnp.dot(z, wc_ref[ci * 32:(ci + 1) * 32, :],
                        preferred_element_type=jnp.float32)
            s = p if s is None else s + p
        out_ref[...] = jax.nn.sigmoid(s + bc_ref[...])

    return pl.pallas_call(
        body,
        grid=(NP // BN,),
        in_specs=[pl.BlockSpec((2, BN, 32), lambda i: (0, i, 0)),
                  pl.BlockSpec((2, BN, 32), lambda i: (0, i, 0)),
                  pl.BlockSpec((BN,), lambda i: (i,)),
                  pl.BlockSpec((64,), lambda i: (0,)),
                  pl.BlockSpec((64, 128), lambda i: (0, 0)),
                  pl.BlockSpec((128,), lambda i: (0,))],
        out_specs=[pl.BlockSpec((BN, 128), lambda i: (i, 0)),
                   pl.BlockSpec((BN, 64), lambda i: (i, 0))],
        out_shape=[jax.ShapeDtypeStruct((NP, 128), jnp.float32),
                   jax.ShapeDtypeStruct((NP, 64), jnp.float32)],
    )(acc3, hs3, dinv, b3p, Wcp, bcp)


def kernel(x, edges, layers_lengths, W1, b1, W2, b2, W3, b3, Wc, bc):
    del layers_lengths
    f32 = jnp.float32
    pad = jnp.full((EP - E,), N, jnp.int32)
    src3 = jnp.concatenate([edges[0], pad]).reshape(ER, 128)
    dst3 = jnp.concatenate([edges[1], pad]).reshape(ER, 128)

    xp = jnp.zeros((NP, 1024), f32).at[:N, :1000].set(x)
    W1p = jnp.zeros((1024, 512), f32).at[:1000].set(W1)
    W3p = jnp.zeros((256, 64), f32).at[:, :52].set(W3)
    b3p = jnp.zeros((64,), f32).at[:52].set(b3)
    Wcp = jnp.zeros((64, 128), f32).at[:52, :3].set(Wc)
    bcp = jnp.zeros((128,), f32).at[:3].set(bc)

    deg16 = _deg(dst3)
    h1 = _matmul(xp, W1p)
    hs1, dinv = _scale_chunk(h1, deg16)
    acc1 = _propagate(hs1, src3, dst3, 8, 64)
    hs2 = _trans(acc1, hs1, dinv, b1, W2, 8, 64, 4, 64)
    acc2 = _propagate(hs2, src3, dst3, 4, 64)
    hs3 = _trans(acc2, hs2, dinv, b2, W3p, 4, 64, 2, 32)
    acc3 = _propagate(hs3, src3, dst3, 2, 32)
    out_full, h_full = _final(acc3, hs3, dinv, b3p, Wcp, bcp)
    return (out_full[:N, :3], h_full[:N, :52])


# speedup vs baseline: 6.0453x; 1.0695x over previous
import functools

import jax
import jax.numpy as jnp
from jax import lax
from jax.experimental import pallas as pl
from jax.experimental.pallas import tpu as pltpu
from jax.experimental.pallas import tpu_sc as plsc

N = 10000
NP = 10240
E = 160000
EP = 163840
ER = EP // 128
NC, NS = 2, 16
RPT = NP // NS
BN = 256

_mesh = plsc.VectorSubcoreMesh(core_axis_name="c", subcore_axis_name="s")
_sc_params = pltpu.CompilerParams(use_tc_tiling_on_sc=False)


def _deg(dst3):
    @functools.partial(
        pl.kernel,
        out_type=jax.ShapeDtypeStruct((NC, NP, 16), jnp.float32),
        mesh=_mesh,
        compiler_params=_sc_params,
        scratch_types=[
            pltpu.VMEM((4, 128), jnp.int32),
            pltpu.VMEM((128, 16), jnp.float32),
            pltpu.VMEM((128, 16), jnp.float32),
            pltpu.VMEM_SHARED((NP, 16), jnp.float32),
        ],
    )
    def k(dst_hbm, deg_hbm, idx_v, ones_v, zero_v, deg_sh):
        core = lax.axis_index("c")
        sid = lax.axis_index("s")

        @pl.loop(0, 128)
        def _(i):
            ones_v[i] = jnp.full((16,), 1.0, jnp.float32)
            zero_v[i] = jnp.zeros((16,), jnp.float32)

        @pl.loop(0, RPT // 128)
        def _(i):
            pltpu.sync_copy(zero_v, deg_sh.at[pl.ds(sid * RPT + i * 128, 128)])

        plsc.subcore_barrier()

        rpt = ER // NC // NS

        @pl.loop(0, rpt // 4)
        def _(r):
            base = core * (ER // NC) + sid * rpt + r * 4
            pltpu.sync_copy(dst_hbm.at[pl.ds(base, 4)], idx_v)
            for j in range(4):
                pltpu.sync_copy(ones_v, deg_sh.at[idx_v.at[j]], add=True)

        plsc.subcore_barrier()

        @pl.loop(0, RPT // 128)
        def _(i):
            s = pl.ds(sid * RPT + i * 128, 128)
            pltpu.sync_copy(deg_sh.at[s], deg_hbm.at[core].at[s])

    return k(dst3)


NBUF = 5


def _propagate(hs, src3, dst3, C, Dc):
    rpt = ER // NS

    @functools.partial(
        pl.kernel,
        out_type=jax.ShapeDtypeStruct((C, NP, Dc), jnp.float32),
        mesh=_mesh,
        compiler_params=_sc_params,
        scratch_types=[
            pltpu.VMEM((rpt, 128), jnp.int32),
            pltpu.VMEM((rpt, 128), jnp.int32),
            pltpu.VMEM((NBUF, 128, Dc), jnp.float32),
            pltpu.VMEM((128, Dc), jnp.float32),
            pltpu.VMEM_SHARED((NP, Dc), jnp.float32),
        ] + [pltpu.SemaphoreType.DMA] * (2 * NBUF),
    )
    def k(hs_hbm, src_hbm, dst_hbm, acc_hbm, src_all, dst_all, ring, zero_v,
          acc_sh, *sems):
        gsem = sems[:NBUF]
        ssem = sems[NBUF:]
        core = lax.axis_index("c")
        sid = lax.axis_index("s")

        @pl.loop(0, 128)
        def _(i):
            for j in range(Dc // 16):
                zero_v[i, pl.ds(j * 16, 16)] = jnp.zeros((16,), jnp.float32)

        pltpu.sync_copy(src_hbm.at[pl.ds(sid * rpt, rpt)], src_all)
        pltpu.sync_copy(dst_hbm.at[pl.ds(sid * rpt, rpt)], dst_all)

        nloops = rpt // NBUF
        for cc in range(C // NC):
            c = cc * NC + core

            @pl.loop(0, RPT // 128)
            def _(i):
                pltpu.sync_copy(zero_v, acc_sh.at[pl.ds(sid * RPT + i * 128, 128)])

            plsc.subcore_barrier()

            for b in range(NBUF):
                pltpu.async_copy(hs_hbm.at[c].at[src_all.at[b]], ring.at[b],
                                 gsem[b])

            @pl.loop(0, nloops - 1)
            def _(r):
                for b in range(NBUF):
                    g = r * NBUF + b
                    pltpu.make_async_copy(hs_hbm.at[c].at[pl.ds(0, 128)],
                                          ring.at[b], gsem[b]).wait()
                    pltpu.async_copy(ring.at[b], acc_sh.at[dst_all.at[g]],
                                     ssem[b], add=True)
                    pltpu.make_async_copy(hs_hbm.at[c].at[pl.ds(0, 128)],
                                          ring.at[b], ssem[b]).wait()
                    pltpu.async_copy(hs_hbm.at[c].at[src_all.at[g + NBUF]],
                                     ring.at[b], gsem[b])

            for b in range(NBUF):
                g = (nloops - 1) * NBUF + b
                pltpu.make_async_copy(hs_hbm.at[c].at[pl.ds(0, 128)],
                                      ring.at[b], gsem[b]).wait()
                pltpu.async_copy(ring.at[b], acc_sh.at[dst_all.at[g]],
                                 ssem[b], add=True)
            for b in range(NBUF):
                pltpu.make_async_copy(hs_hbm.at[c].at[pl.ds(0, 128)],
                                      ring.at[b], ssem[b]).wait()

            plsc.subcore_barrier()
            pltpu.sync_copy(acc_sh.at[pl.ds(sid * RPT, RPT)],
                            acc_hbm.at[c].at[pl.ds(sid * RPT, RPT)])

    return k(hs, src3, dst3)


def _matmul(x, Wp):
    K, Do = Wp.shape

    def body(x_ref, w_ref, o_ref):
        o_ref[...] = jnp.dot(x_ref[...], w_ref[...],
                             preferred_element_type=jnp.float32)

    return pl.pallas_call(
        body,
        grid=(NP // BN,),
        in_specs=[pl.BlockSpec((BN, K), lambda i: (i, 0)),
                  pl.BlockSpec((K, Do), lambda i: (0, 0))],
        out_specs=pl.BlockSpec((BN, Do), lambda i: (i, 0)),
        out_shape=jax.ShapeDtypeStruct((NP, Do), jnp.float32),
    )(x, Wp)


def _scale_chunk(h1, deg16):
    def body(h_ref, deg_ref, hs_ref, dinv_ref):
        dg = deg_ref[...]
        deg = dg[0, :, 0] + dg[1, :, 0] + 1.0
        dinv = lax.rsqrt(deg)
        dinv_ref[...] = dinv
        hs = h_ref[...] * dinv[:, None]
        for ci in range(8):
            hs_ref[ci] = hs[:, ci * 64:(ci + 1) * 64]

    return pl.pallas_call(
        body,
        grid=(NP // BN,),
        in_specs=[pl.BlockSpec((BN, 512), lambda i: (i, 0)),
                  pl.BlockSpec((2, BN, 16), lambda i: (0, i, 0))],
        out_specs=[pl.BlockSpec((8, BN, 64), lambda i: (0, i, 0)),
                   pl.BlockSpec((BN,), lambda i: (i,))],
        out_shape=[jax.ShapeDtypeStruct((8, NP, 64), jnp.float32),
                   jax.ShapeDtypeStruct((NP,), jnp.float32)],
    )(h1, deg16)


def _trans(acc, hs, dinv, b, W, C, Dc, C2, Dc2):
    dk = C * Dc
    dn = W.shape[1]

    def body(acc_ref, hs_ref, dinv_ref, b_ref, w_ref, o_ref):
        dinv = dinv_ref[...]
        s = None
        for ci in range(C):
            z = jnp.clip(dinv[:, None] * (acc_ref[ci] + hs_ref[ci])
                         + b_ref[ci * Dc:(ci + 1) * Dc], 0.0, 6.0)
            p = jnp.dot(z, w_ref[ci * Dc:(ci + 1) * Dc, :],
                        preferred_element_type=jnp.float32)
            s = p if s is None else s + p
        hsn = s * dinv[:, None]
        for ci in range(C2):
            o_ref[ci] = hsn[:, ci * Dc2:(ci + 1) * Dc2]

    return pl.pallas_call(
        body,
        grid=(NP // BN,),
        in_specs=[pl.BlockSpec((C, BN, Dc), lambda i: (0, i, 0)),
                  pl.BlockSpec((C, BN, Dc), lambda i: (0, i, 0)),
                  pl.BlockSpec((BN,), lambda i: (i,)),
                  pl.BlockSpec((dk,), lambda i: (0,)),
                  pl.BlockSpec((dk, dn), lambda i: (0, 0))],
        out_specs=pl.BlockSpec((C2, BN, Dc2), lambda i: (0, i, 0)),
        out_shape=jax.ShapeDtypeStruct((C2, NP, Dc2), jnp.float32),
    )(acc, hs, dinv, b, W)


def _final(acc3, hs3, dinv, b3p, Wcp, bcp):
    def body(acc_ref, hs_ref, dinv_ref, b_ref, wc_ref, bc_ref, out_ref, h_ref):
        dinv = dinv_ref[...]
        s = None
        for ci in range(2):
            z = jnp.clip(dinv[:, None] * (acc_ref[ci] + hs_ref[ci])
                         + b_ref[ci * 32:(ci + 1) * 32], 0.0, 6.0)
            h_ref[:, ci * 32:(ci + 1) * 32] = z
            p = jnp.dot(z, wc_ref[ci * 32:(ci + 1) * 32, :],
                        preferred_element_type=jnp.float32)
            s = p if s is None else s + p
        out_ref[...] = jax.nn.sigmoid(s + bc_ref[...])

    return pl.pallas_call(
        body,
        grid=(NP // BN,),
        in_specs=[pl.BlockSpec((2, BN, 32), lambda i: (0, i, 0)),
                  pl.BlockSpec((2, BN, 32), lambda i: (0, i, 0)),
                  pl.BlockSpec((BN,), lambda i: (i,)),
                  pl.BlockSpec((64,), lambda i: (0,)),
                  pl.BlockSpec((64, 128), lambda i: (0, 0)),
                  pl.BlockSpec((128,), lambda i: (0,))],
        out_specs=[pl.BlockSpec((BN, 128), lambda i: (i, 0)),
                   pl.BlockSpec((BN, 64), lambda i: (i, 0))],
        out_shape=[jax.ShapeDtypeStruct((NP, 128), jnp.float32),
                   jax.ShapeDtypeStruct((NP, 64), jnp.float32)],
    )(acc3, hs3, dinv, b3p, Wcp, bcp)


def kernel(x, edges, layers_lengths, W1, b1, W2, b2, W3, b3, Wc, bc):
    del layers_lengths
    f32 = jnp.float32
    pad = jnp.full((EP - E,), N, jnp.int32)
    src3 = jnp.concatenate([edges[0], pad]).reshape(ER, 128)
    dst3 = jnp.concatenate([edges[1], pad]).reshape(ER, 128)

    W3p = jnp.zeros((256, 64), f32).at[:, :52].set(W3)
    b3p = jnp.zeros((64,), f32).at[:52].set(b3)
    Wcp = jnp.zeros((64, 128), f32).at[:52, :3].set(Wc)
    bcp = jnp.zeros((128,), f32).at[:3].set(bc)

    deg16 = _deg(dst3)
    h1 = _matmul(x, W1)
    hs1, dinv = _scale_chunk(h1, deg16)
    acc1 = _propagate(hs1, src3, dst3, 8, 64)
    hs2 = _trans(acc1, hs1, dinv, b1, W2, 8, 64, 4, 64)
    acc2 = _propagate(hs2, src3, dst3, 4, 64)
    hs3 = _trans(acc2, hs2, dinv, b2, W3p, 4, 64, 2, 32)
    acc3 = _propagate(hs3, src3, dst3, 2, 32)
    out_full, h_full = _final(acc3, hs3, dinv, b3p, Wcp, bcp)
    return (out_full[:N, :3], h_full[:N, :52])


# speedup vs baseline: 10.5706x; 1.7486x over previous
import functools

import jax
import jax.numpy as jnp
from jax import lax
from jax.experimental import pallas as pl
from jax.experimental.pallas import tpu as pltpu
from jax.experimental.pallas import tpu_sc as plsc

N = 10000
NP = 10240
E = 160000
EP = 163840
ER = EP // 128
NC, NS = 2, 16
RPT = NP // NS
BN = 256
NBUF = 5

_mesh = plsc.VectorSubcoreMesh(core_axis_name="c", subcore_axis_name="s")
_sc_params = pltpu.CompilerParams(use_tc_tiling_on_sc=False)


def _deg(dst3):
    @functools.partial(
        pl.kernel,
        out_type=jax.ShapeDtypeStruct((NC, NP, 16), jnp.float32),
        mesh=_mesh,
        compiler_params=_sc_params,
        scratch_types=[
            pltpu.VMEM((4, 128), jnp.int32),
            pltpu.VMEM((128, 16), jnp.float32),
            pltpu.VMEM((128, 16), jnp.float32),
            pltpu.VMEM_SHARED((NP, 16), jnp.float32),
        ],
    )
    def k(dst_hbm, deg_hbm, idx_v, ones_v, zero_v, deg_sh):
        core = lax.axis_index("c")
        sid = lax.axis_index("s")

        @pl.loop(0, 128)
        def _(i):
            ones_v[i] = jnp.full((16,), 1.0, jnp.float32)
            zero_v[i] = jnp.zeros((16,), jnp.float32)

        @pl.loop(0, RPT // 128)
        def _(i):
            pltpu.sync_copy(zero_v, deg_sh.at[pl.ds(sid * RPT + i * 128, 128)])

        plsc.subcore_barrier()

        rpt = ER // NC // NS

        @pl.loop(0, rpt // 4)
        def _(r):
            base = core * (ER // NC) + sid * rpt + r * 4
            pltpu.sync_copy(dst_hbm.at[pl.ds(base, 4)], idx_v)
            for j in range(4):
                pltpu.sync_copy(ones_v, deg_sh.at[idx_v.at[j]], add=True)

        plsc.subcore_barrier()

        @pl.loop(0, RPT // 128)
        def _(i):
            s = pl.ds(sid * RPT + i * 128, 128)
            pltpu.sync_copy(deg_sh.at[s], deg_hbm.at[core].at[s])

    return k(dst3)


def _propagate(hs, src3, dst3, C):
    D = hs.shape[1]
    rpt = ER // NS

    @functools.partial(
        pl.kernel,
        out_type=jax.ShapeDtypeStruct((NP, D), jnp.float32),
        mesh=_mesh,
        compiler_params=_sc_params,
        scratch_types=[
            pltpu.VMEM((rpt, 128), jnp.int32),
            pltpu.VMEM((rpt, 128), jnp.int32),
            pltpu.VMEM((NBUF, 128, 32), jnp.float32),
            pltpu.VMEM((128, 32), jnp.float32),
            pltpu.VMEM_SHARED((NP, 32), jnp.float32),
            pltpu.VMEM_SHARED((NP, 32), jnp.float32),
        ] + [pltpu.SemaphoreType.DMA] * (2 * NBUF),
    )
    def k(hs_hbm, src_hbm, dst_hbm, acc_hbm, src_all, dst_all, ring, zero_v,
          tab_sh, acc_sh, *sems):
        gsem = sems[:NBUF]
        ssem = sems[NBUF:]
        core = lax.axis_index("c")
        sid = lax.axis_index("s")

        @pl.loop(0, 128)
        def _(i):
            for j in range(2):
                zero_v[i, pl.ds(j * 16, 16)] = jnp.zeros((16,), jnp.float32)

        pltpu.sync_copy(src_hbm.at[pl.ds(sid * rpt, rpt)], src_all)
        pltpu.sync_copy(dst_hbm.at[pl.ds(sid * rpt, rpt)], dst_all)

        nloops = rpt // NBUF
        rows = pl.ds(sid * RPT, RPT)
        for cc in range(C // NC):
            c = cc * NC + core
            cols = pl.ds(c * 32, 32)

            pltpu.sync_copy(hs_hbm.at[rows, cols], tab_sh.at[rows])

            @pl.loop(0, RPT // 128)
            def _(i):
                pltpu.sync_copy(zero_v, acc_sh.at[pl.ds(sid * RPT + i * 128, 128)])

            plsc.subcore_barrier()

            for b in range(NBUF):
                pltpu.async_copy(tab_sh.at[src_all.at[b]], ring.at[b],
                                 gsem[b])

            @pl.loop(0, nloops - 1)
            def _(r):
                for b in range(NBUF):
                    g = r * NBUF + b
                    pltpu.make_async_copy(tab_sh.at[pl.ds(0, 128)],
                                          ring.at[b], gsem[b]).wait()
                    pltpu.async_copy(ring.at[b], acc_sh.at[dst_all.at[g]],
                                     ssem[b], add=True)
                    pltpu.make_async_copy(tab_sh.at[pl.ds(0, 128)],
                                          ring.at[b], ssem[b]).wait()
                    pltpu.async_copy(tab_sh.at[src_all.at[g + NBUF]],
                                     ring.at[b], gsem[b])

            for b in range(NBUF):
                g = (nloops - 1) * NBUF + b
                pltpu.make_async_copy(tab_sh.at[pl.ds(0, 128)],
                                      ring.at[b], gsem[b]).wait()
                pltpu.async_copy(ring.at[b], acc_sh.at[dst_all.at[g]],
                                 ssem[b], add=True)
            for b in range(NBUF):
                pltpu.make_async_copy(tab_sh.at[pl.ds(0, 128)],
                                      ring.at[b], ssem[b]).wait()

            plsc.subcore_barrier()
            pltpu.sync_copy(acc_sh.at[rows], acc_hbm.at[rows, cols])

    return k(hs, src3, dst3)


def _m1(x, W1, deg16):
    def body(x_ref, w_ref, deg_ref, hs_ref, dinv_ref):
        dg = deg_ref[...]
        dinv = lax.rsqrt(dg[0, :, 0] + dg[1, :, 0] + 1.0)
        dinv_ref[...] = dinv
        h = jnp.dot(x_ref[...], w_ref[...], preferred_element_type=jnp.float32)
        hs_ref[...] = h * dinv[:, None]

    return pl.pallas_call(
        body,
        grid=(NP // BN,),
        in_specs=[pl.BlockSpec((BN, 1000), lambda i: (i, 0)),
                  pl.BlockSpec((1000, 512), lambda i: (0, 0)),
                  pl.BlockSpec((2, BN, 16), lambda i: (0, i, 0))],
        out_specs=[pl.BlockSpec((BN, 512), lambda i: (i, 0)),
                   pl.BlockSpec((BN,), lambda i: (i,))],
        out_shape=[jax.ShapeDtypeStruct((NP, 512), jnp.float32),
                   jax.ShapeDtypeStruct((NP,), jnp.float32)],
    )(x, W1, deg16)


def _trans(acc, hs, dinv, b, W):
    dk = W.shape[0]
    dn = W.shape[1]

    def body(acc_ref, hs_ref, dinv_ref, b_ref, w_ref, o_ref):
        dinv = dinv_ref[...]
        z = jnp.clip(dinv[:, None] * (acc_ref[...] + hs_ref[...])
                     + b_ref[...], 0.0, 6.0)
        o_ref[...] = jnp.dot(z, w_ref[...],
                             preferred_element_type=jnp.float32) * dinv[:, None]

    return pl.pallas_call(
        body,
        grid=(NP // BN,),
        in_specs=[pl.BlockSpec((BN, dk), lambda i: (i, 0)),
                  pl.BlockSpec((BN, dk), lambda i: (i, 0)),
                  pl.BlockSpec((BN,), lambda i: (i,)),
                  pl.BlockSpec((dk,), lambda i: (0,)),
                  pl.BlockSpec((dk, dn), lambda i: (0, 0))],
        out_specs=pl.BlockSpec((BN, dn), lambda i: (i, 0)),
        out_shape=jax.ShapeDtypeStruct((NP, dn), jnp.float32),
    )(acc, hs, dinv, b, W)


def _final(acc3, hs3, dinv, b3p, Wcp, bcp):
    def body(acc_ref, hs_ref, dinv_ref, b_ref, wc_ref, bc_ref, out_ref, h_ref):
        dinv = dinv_ref[...]
        z = jnp.clip(dinv[:, None] * (acc_ref[...][:, :64] + hs_ref[...][:, :64])
                     + b_ref[...], 0.0, 6.0)
        h_ref[...] = z
        p = jnp.dot(z, wc_ref[...], preferred_element_type=jnp.float32)
        out_ref[...] = jax.nn.sigmoid(p + bc_ref[...])

    return pl.pallas_call(
        body,
        grid=(NP // BN,),
        in_specs=[pl.BlockSpec((BN, 128), lambda i: (i, 0)),
                  pl.BlockSpec((BN, 128), lambda i: (i, 0)),
                  pl.BlockSpec((BN,), lambda i: (i,)),
                  pl.BlockSpec((64,), lambda i: (0,)),
                  pl.BlockSpec((64, 128), lambda i: (0, 0)),
                  pl.BlockSpec((128,), lambda i: (0,))],
        out_specs=[pl.BlockSpec((BN, 128), lambda i: (i, 0)),
                   pl.BlockSpec((BN, 64), lambda i: (i, 0))],
        out_shape=[jax.ShapeDtypeStruct((NP, 128), jnp.float32),
                   jax.ShapeDtypeStruct((NP, 64), jnp.float32)],
    )(acc3, hs3, dinv, b3p, Wcp, bcp)


def kernel(x, edges, layers_lengths, W1, b1, W2, b2, W3, b3, Wc, bc):
    del layers_lengths
    f32 = jnp.float32
    pad = jnp.full((EP - E,), N, jnp.int32)
    src3 = jnp.concatenate([edges[0], pad]).reshape(ER, 128)
    dst3 = jnp.concatenate([edges[1], pad]).reshape(ER, 128)

    W3p = jnp.zeros((256, 128), f32).at[:, :52].set(W3)
    b3p = jnp.zeros((64,), f32).at[:52].set(b3)
    Wcp = jnp.zeros((64, 128), f32).at[:52, :3].set(Wc)
    bcp = jnp.zeros((128,), f32).at[:3].set(bc)

    deg16 = _deg(dst3)
    hs1, dinv = _m1(x, W1, deg16)
    acc1 = _propagate(hs1, src3, dst3, 16)
    hs2 = _trans(acc1, hs1, dinv, b1, W2)
    acc2 = _propagate(hs2, src3, dst3, 8)
    hs3 = _trans(acc2, hs2, dinv, b2, W3p)
    acc3 = _propagate(hs3, src3, dst3, 2)
    out_full, h_full = _final(acc3, hs3, dinv, b3p, Wcp, bcp)
    return (out_full[:N, :3], h_full[:N, :52])


# speedup vs baseline: 10.8412x; 1.0256x over previous
import functools

import jax
import jax.numpy as jnp
from jax import lax
from jax.experimental import pallas as pl
from jax.experimental.pallas import tpu as pltpu
from jax.experimental.pallas import tpu_sc as plsc

N = 10000
NP = 10240
E = 160000
EP = 163840
ER = EP // 128
NC, NS = 2, 16
RPT = NP // NS
BN = 256
NBUF = 5

_mesh = plsc.VectorSubcoreMesh(core_axis_name="c", subcore_axis_name="s")
_sc_params = pltpu.CompilerParams(use_tc_tiling_on_sc=False)
_sc_params_tiled = pltpu.CompilerParams(use_tc_tiling_on_sc=True)


def _deg(dst3):
    @functools.partial(
        pl.kernel,
        out_type=jax.ShapeDtypeStruct((NC, NP, 16), jnp.float32),
        mesh=_mesh,
        compiler_params=_sc_params,
        scratch_types=[
            pltpu.VMEM((4, 128), jnp.int32),
            pltpu.VMEM((128, 16), jnp.float32),
            pltpu.VMEM((128, 16), jnp.float32),
            pltpu.VMEM_SHARED((NP, 16), jnp.float32),
        ],
    )
    def k(dst_hbm, deg_hbm, idx_v, ones_v, zero_v, deg_sh):
        core = lax.axis_index("c")
        sid = lax.axis_index("s")

        @pl.loop(0, 128)
        def _(i):
            ones_v[i] = jnp.full((16,), 1.0, jnp.float32)
            zero_v[i] = jnp.zeros((16,), jnp.float32)

        @pl.loop(0, RPT // 128)
        def _(i):
            pltpu.sync_copy(zero_v, deg_sh.at[pl.ds(sid * RPT + i * 128, 128)])

        plsc.subcore_barrier()

        rpt = ER // NC // NS

        @pl.loop(0, rpt // 4)
        def _(r):
            base = core * (ER // NC) + sid * rpt + r * 4
            pltpu.sync_copy(dst_hbm.at[pl.ds(base, 4)], idx_v)
            for j in range(4):
                pltpu.sync_copy(ones_v, deg_sh.at[idx_v.at[j]], add=True)

        plsc.subcore_barrier()

        @pl.loop(0, RPT // 128)
        def _(i):
            s = pl.ds(sid * RPT + i * 128, 128)
            pltpu.sync_copy(deg_sh.at[s], deg_hbm.at[core].at[s])

    return k(dst3)


def _propagate(hs, src3, dst3, C):
    D = hs.shape[1]
    rpt = ER // NS

    @functools.partial(
        pl.kernel,
        out_type=jax.ShapeDtypeStruct((NP, D), jnp.float32),
        mesh=_mesh,
        compiler_params=_sc_params,
        scratch_types=[
            pltpu.VMEM((rpt, 128), jnp.int32),
            pltpu.VMEM((rpt, 128), jnp.int32),
            pltpu.VMEM((NBUF, 128, 32), jnp.float32),
            pltpu.VMEM((128, 32), jnp.float32),
            pltpu.VMEM_SHARED((NP, 32), jnp.float32),
            pltpu.VMEM_SHARED((NP, 32), jnp.float32),
        ] + [pltpu.SemaphoreType.DMA] * (2 * NBUF),
    )
    def k(hs_hbm, src_hbm, dst_hbm, acc_hbm, src_all, dst_all, ring, zero_v,
          tab_sh, acc_sh, *sems):
        gsem = sems[:NBUF]
        ssem = sems[NBUF:]
        core = lax.axis_index("c")
        sid = lax.axis_index("s")

        @pl.loop(0, 128)
        def _(i):
            for j in range(2):
                zero_v[i, pl.ds(j * 16, 16)] = jnp.zeros((16,), jnp.float32)

        pltpu.sync_copy(src_hbm.at[pl.ds(sid * rpt, rpt)], src_all)
        pltpu.sync_copy(dst_hbm.at[pl.ds(sid * rpt, rpt)], dst_all)

        nloops = rpt // NBUF
        rows = pl.ds(sid * RPT, RPT)
        for cc in range(C // NC):
            c = cc * NC + core
            cols = pl.ds(c * 32, 32)

            pltpu.sync_copy(hs_hbm.at[rows, cols], tab_sh.at[rows])

            @pl.loop(0, RPT // 128)
            def _(i):
                pltpu.sync_copy(zero_v, acc_sh.at[pl.ds(sid * RPT + i * 128, 128)])

            plsc.subcore_barrier()

            for b in range(NBUF):
                pltpu.async_copy(tab_sh.at[src_all.at[b]], ring.at[b],
                                 gsem[b])

            @pl.loop(0, nloops - 1)
            def _(r):
                for b in range(NBUF):
                    g = r * NBUF + b
                    pltpu.make_async_copy(tab_sh.at[pl.ds(0, 128)],
                                          ring.at[b], gsem[b]).wait()
                    pltpu.async_copy(ring.at[b], acc_sh.at[dst_all.at[g]],
                                     ssem[b], add=True)
                    pltpu.make_async_copy(tab_sh.at[pl.ds(0, 128)],
                                          ring.at[b], ssem[b]).wait()
                    pltpu.async_copy(tab_sh.at[src_all.at[g + NBUF]],
                                     ring.at[b], gsem[b])

            for b in range(NBUF):
                g = (nloops - 1) * NBUF + b
                pltpu.make_async_copy(tab_sh.at[pl.ds(0, 128)],
                                      ring.at[b], gsem[b]).wait()
                pltpu.async_copy(ring.at[b], acc_sh.at[dst_all.at[g]],
                                 ssem[b], add=True)
            for b in range(NBUF):
                pltpu.make_async_copy(tab_sh.at[pl.ds(0, 128)],
                                      ring.at[b], ssem[b]).wait()

            plsc.subcore_barrier()
            pltpu.sync_copy(acc_sh.at[rows], acc_hbm.at[rows, cols])

    return k(hs, src3, dst3)


def _m1(xt, W1, deg16):
    def body(xt_ref, w_ref, deg_ref, hs_ref, dinv_ref):
        dg = deg_ref[...]
        dinv = lax.rsqrt(dg[0, :, 0] + dg[1, :, 0] + 1.0)
        dinv_ref[...] = dinv
        h = lax.dot_general(xt_ref[...], w_ref[...],
                            dimension_numbers=(((0,), (0,)), ((), ())),
                            preferred_element_type=jnp.float32)
        hs_ref[...] = h * dinv[:, None]

    return pl.pallas_call(
        body,
        grid=(NP // BN,),
        in_specs=[pl.BlockSpec((1000, BN), lambda i: (0, i)),
                  pl.BlockSpec((1000, 512), lambda i: (0, 0)),
                  pl.BlockSpec((2, BN, 16), lambda i: (0, i, 0))],
        out_specs=[pl.BlockSpec((BN, 512), lambda i: (i, 0)),
                   pl.BlockSpec((BN,), lambda i: (i,))],
        out_shape=[jax.ShapeDtypeStruct((NP, 512), jnp.float32),
                   jax.ShapeDtypeStruct((NP,), jnp.float32)],
    )(xt, W1, deg16)


def _trans(acc, hs, dinv, b, W):
    dk = W.shape[0]
    dn = W.shape[1]

    def body(acc_ref, hs_ref, dinv_ref, b_ref, w_ref, o_ref):
        dinv = dinv_ref[...]
        z = jnp.clip(dinv[:, None] * (acc_ref[...] + hs_ref[...])
                     + b_ref[...], 0.0, 6.0)
        o_ref[...] = jnp.dot(z, w_ref[...],
                             preferred_element_type=jnp.float32) * dinv[:, None]

    return pl.pallas_call(
        body,
        grid=(NP // BN,),
        in_specs=[pl.BlockSpec((BN, dk), lambda i: (i, 0)),
                  pl.BlockSpec((BN, dk), lambda i: (i, 0)),
                  pl.BlockSpec((BN,), lambda i: (i,)),
                  pl.BlockSpec((dk,), lambda i: (0,)),
                  pl.BlockSpec((dk, dn), lambda i: (0, 0))],
        out_specs=pl.BlockSpec((BN, dn), lambda i: (i, 0)),
        out_shape=jax.ShapeDtypeStruct((NP, dn), jnp.float32),
    )(acc, hs, dinv, b, W)


def _final(acc3, hs3, dinv, b3p, Wcp, bcp):
    def body(acc_ref, hs_ref, dinv_ref, b_ref, wc_ref, bc_ref, out_ref, h_ref):
        dinv = dinv_ref[...]
        z = jnp.clip(dinv[:, None] * (acc_ref[...][:, :64] + hs_ref[...][:, :64])
                     + b_ref[...], 0.0, 6.0)
        h_ref[...] = z
        p = jnp.dot(z, wc_ref[...], preferred_element_type=jnp.float32)
        out_ref[...] = jax.nn.sigmoid(p + bc_ref[...])

    return pl.pallas_call(
        body,
        grid=(NP // BN,),
        in_specs=[pl.BlockSpec((BN, 128), lambda i: (i, 0)),
                  pl.BlockSpec((BN, 128), lambda i: (i, 0)),
                  pl.BlockSpec((BN,), lambda i: (i,)),
                  pl.BlockSpec((64,), lambda i: (0,)),
                  pl.BlockSpec((64, 128), lambda i: (0, 0)),
                  pl.BlockSpec((128,), lambda i: (0,))],
        out_specs=[pl.BlockSpec((BN, 128), lambda i: (i, 0)),
                   pl.BlockSpec((BN, 64), lambda i: (i, 0))],
        out_shape=[jax.ShapeDtypeStruct((NP, 128), jnp.float32),
                   jax.ShapeDtypeStruct((NP, 64), jnp.float32)],
    )(acc3, hs3, dinv, b3p, Wcp, bcp)


def kernel(x, edges, layers_lengths, W1, b1, W2, b2, W3, b3, Wc, bc):
    del layers_lengths
    f32 = jnp.float32
    pad = jnp.full((EP - E,), N, jnp.int32)
    src3 = jnp.concatenate([edges[0], pad]).reshape(ER, 128)
    dst3 = jnp.concatenate([edges[1], pad]).reshape(ER, 128)

    W3p = jnp.zeros((256, 128), f32).at[:, :52].set(W3)
    b3p = jnp.zeros((64,), f32).at[:52].set(b3)
    Wcp = jnp.zeros((64, 128), f32).at[:52, :3].set(Wc)
    bcp = jnp.zeros((128,), f32).at[:3].set(bc)

    deg16 = _deg(dst3)
    hs1, dinv = _m1(jnp.transpose(x), W1, deg16)
    acc1 = _propagate(hs1, src3, dst3, 16)
    hs2 = _trans(acc1, hs1, dinv, b1, W2)
    acc2 = _propagate(hs2, src3, dst3, 8)
    hs3 = _trans(acc2, hs2, dinv, b2, W3p)
    acc3 = _propagate(hs3, src3, dst3, 2)
    out_full, h_full = _final(acc3, hs3, dinv, b3p, Wcp, bcp)
    return (out_full[:N, :3], h_full[:N, :52])
